# Initial kernel scaffold; baseline (speedup 1.0000x reference)
import jax
import jax.numpy as jnp
from jax.experimental import pallas as pl


def kernel(features, edge_index, W, b, gamma, beta):
    raise NotImplementedError("write your pallas kernel here")


# speedup vs baseline: 7.6520x; 7.6520x over previous
import functools

import jax
import jax.numpy as jnp
from jax import lax
from jax.experimental import pallas as pl
from jax.experimental.pallas import tpu as pltpu
from jax.experimental.pallas import tpu_sc as plsc

_NCORE = 2
_NSUB = 16
_NW = _NCORE * _NSUB
_C = 128


def _sc_mesh():
    return plsc.VectorSubcoreMesh(core_axis_name="c", subcore_axis_name="s")


def _sc_degree_rowmap(row_p, col_p, n_nodes, np_nodes, kc):
    epad = row_p.shape[0]
    rt = np_nodes // _NSUB

    @functools.partial(
        pl.kernel,
        out_type=(
            jax.ShapeDtypeStruct((epad,), jnp.int32),
            jax.ShapeDtypeStruct((_NCORE, np_nodes), jnp.float32),
        ),
        mesh=_sc_mesh(),
        scratch_types=[
            pltpu.VMEM((_C,), jnp.int32),
            pltpu.VMEM((_C,), jnp.int32),
            pltpu.VMEM((_C,), jnp.int32),
            pltpu.VMEM((_C,), jnp.float32),
            pltpu.VMEM((rt,), jnp.float32),
            pltpu.VMEM_SHARED((np_nodes,), jnp.float32),
        ],
    )
    def k(row_hbm, col_hbm, rowmap_hbm, degp_hbm, rbuf, cbuf, rmbuf, wbuf,
          stage, degacc):
        core = lax.axis_index("c")
        sub = lax.axis_index("s")
        wid = core * _NSUB + sub

        zeros16 = jnp.zeros((16,), jnp.float32)
        ones16 = jnp.ones((16,), jnp.float32)
        dummy16 = jnp.full((16,), n_nodes, jnp.int32)

        def zfill(i, _):
            stage[pl.ds(i * 16, 16)] = zeros16
            return 0

        lax.fori_loop(0, rt // 16, zfill, 0)
        pltpu.sync_copy(stage, degacc.at[pl.ds(sub * rt, rt)])
        plsc.subcore_barrier()

        def body(t, _):
            base = (wid * kc + t) * _C
            pltpu.sync_copy(row_hbm.at[pl.ds(base, _C)], rbuf)
            pltpu.sync_copy(col_hbm.at[pl.ds(base, _C)], cbuf)
            for j in range(_C // 16):
                sl = pl.ds(j * 16, 16)
                r = rbuf[sl]
                m = r == cbuf[sl]
                rmbuf[sl] = jnp.where(m, dummy16, r)
                wbuf[sl] = jnp.where(m, zeros16, ones16)
            pltpu.sync_copy(rmbuf, rowmap_hbm.at[pl.ds(base, _C)])
            pltpu.sync_copy(wbuf, degacc.at[rbuf], add=True)
            return 0

        lax.fori_loop(0, kc, body, 0)
        plsc.subcore_barrier()
        pltpu.sync_copy(degacc.at[pl.ds(sub * rt, rt)], stage)
        pltpu.sync_copy(stage, degp_hbm.at[core, pl.ds(sub * rt, rt)])

    return k(row_p, col_p)


def _sc_propagate(z_pad, rowmap, col_p, np_nodes, kc):
    d = z_pad.shape[1]
    rt = np_nodes // _NSUB

    @functools.partial(
        pl.kernel,
        out_type=jax.ShapeDtypeStruct((_NCORE, np_nodes, d), jnp.float32),
        mesh=_sc_mesh(),
        scratch_types=[
            pltpu.VMEM((_C,), jnp.int32),
            pltpu.VMEM((_C,), jnp.int32),
            pltpu.VMEM((_C, d), jnp.float32),
            pltpu.VMEM((_C, d), jnp.float32),
            pltpu.VMEM_SHARED((np_nodes, d), jnp.float32),
            pltpu.SemaphoreType.DMA,
        ],
    )
    def k(z_hbm, rmap_hbm, col_hbm, out_hbm, ridx, cidx, rows, stage, acc, sem):
        core = lax.axis_index("c")
        sub = lax.axis_index("s")
        wid = core * _NSUB + sub

        zeros16 = jnp.zeros((16,), jnp.float32)

        def zfill(i, _):
            for j in range(d // 16):
                stage[i, pl.ds(j * 16, 16)] = zeros16
            return 0

        lax.fori_loop(0, _C, zfill, 0)
        for i in range(rt // _C):
            pltpu.sync_copy(stage, acc.at[pl.ds(sub * rt + i * _C, _C)])
        plsc.subcore_barrier()

        def body(t, _):
            base = (wid * kc + t) * _C
            pltpu.sync_copy(rmap_hbm.at[pl.ds(base, _C)], ridx)
            pltpu.sync_copy(col_hbm.at[pl.ds(base, _C)], cidx)
            pltpu.async_copy(z_hbm.at[ridx], rows, sem).wait()
            pltpu.sync_copy(rows, acc.at[cidx], add=True)
            return 0

        lax.fori_loop(0, kc, body, 0)
        plsc.subcore_barrier()
        for i in range(rt // _C):
            r0 = sub * rt + i * _C
            pltpu.sync_copy(acc.at[pl.ds(r0, _C)], stage)
            pltpu.sync_copy(stage, out_hbm.at[core, pl.ds(r0, _C)])

    return k(z_pad, rowmap, col_p)


def _tc_prescale(deg4, x_pad):
    npn, d = x_pad.shape
    nb = npn // 128

    def body(degr, xr, dinvbr, z1r):
        dv = degr[0, 0] + degr[1, 0]
        pos = dv > 0.0
        dinv = jnp.where(pos, lax.rsqrt(jnp.where(pos, dv, 1.0)), 0.0)
        rid = lax.broadcasted_iota(jnp.int32, (128, 128), 0)
        cid = lax.broadcasted_iota(jnp.int32, (128, 128), 1)
        diag = jnp.where(rid == cid, jnp.broadcast_to(dinv, (128, 128)), 0.0)
        ones = jnp.ones((128, 128), jnp.float32)
        dm = jnp.dot(diag, ones, preferred_element_type=jnp.float32)
        dinvbr[...] = dm
        z1r[...] = dm * xr[...]

    return pl.pallas_call(
        body,
        grid=(nb,),
        in_specs=[
            pl.BlockSpec((2, 1, 1, 128), lambda g: (0, g, 0, 0)),
            pl.BlockSpec((128, d), lambda g: (g, 0)),
        ],
        out_specs=[
            pl.BlockSpec((128, 128), lambda g: (g, 0)),
            pl.BlockSpec((128, d), lambda g: (g, 0)),
        ],
        out_shape=(
            jax.ShapeDtypeStruct((npn, 128), jnp.float32),
            jax.ShapeDtypeStruct((npn, d), jnp.float32),
        ),
    )(deg4, x_pad)


def _tc_mid(a1, dinvb):
    _, npn, d = a1.shape
    nb = npn // 128

    def body(ar, dr, u1r, z2r):
        dm = dr[...]
        u1 = -(dm * (ar[0] + ar[1]))
        u1r[...] = u1
        z2r[...] = dm * u1

    return pl.pallas_call(
        body,
        grid=(nb,),
        in_specs=[
            pl.BlockSpec((2, 128, d), lambda g: (0, g, 0)),
            pl.BlockSpec((128, 128), lambda g: (g, 0)),
        ],
        out_specs=[
            pl.BlockSpec((128, d), lambda g: (g, 0)),
            pl.BlockSpec((128, d), lambda g: (g, 0)),
        ],
        out_shape=(
            jax.ShapeDtypeStruct((npn, d), jnp.float32),
            jax.ShapeDtypeStruct((npn, d), jnp.float32),
        ),
    )(a1, dinvb)


def _tc_final(x, u1, a2, dinvb, wa, wb, wc, params):
    n, d = x.shape
    rb = 400
    nb = n // rb

    def body(xr, u1r, ar, dr, war, wbr, wcr, pr, outr):
        u2 = dr[...] * (ar[0] + ar[1])
        acc = jnp.dot(xr[...], war[...], preferred_element_type=jnp.float32)
        acc += jnp.dot(u1r[...], wbr[...], preferred_element_type=jnp.float32)
        acc += jnp.dot(u2, wcr[...], preferred_element_type=jnp.float32)
        acc += pr[0:1, :]
        mu = jnp.mean(acc, axis=-1, keepdims=True)
        var = jnp.mean((acc - mu) ** 2, axis=-1, keepdims=True)
        outr[...] = (acc - mu) / jnp.sqrt(var + 1e-5) * pr[1:2, :] + pr[2:3, :]

    return pl.pallas_call(
        body,
        grid=(nb,),
        in_specs=[
            pl.BlockSpec((rb, d), lambda g: (g, 0)),
            pl.BlockSpec((rb, d), lambda g: (g, 0)),
            pl.BlockSpec((2, rb, d), lambda g: (0, g, 0)),
            pl.BlockSpec((rb, 128), lambda g: (g, 0)),
            pl.BlockSpec((d, d), lambda g: (0, 0)),
            pl.BlockSpec((d, d), lambda g: (0, 0)),
            pl.BlockSpec((d, d), lambda g: (0, 0)),
            pl.BlockSpec((8, d), lambda g: (0, 0)),
        ],
        out_specs=pl.BlockSpec((rb, d), lambda g: (g, 0)),
        out_shape=jax.ShapeDtypeStruct((n, d), jnp.float32),
    )(x, u1, a2, dinvb, wa, wb, wc, params)


def kernel(features, edge_index, W, b, gamma, beta):
    n, d = features.shape
    e = edge_index.shape[1]
    npn = -(-n // (_NSUB * _C)) * (_NSUB * _C)
    ept = -(-e // (_NW * _C)) * _C
    kc = ept // _C
    epad = _NW * ept

    row = edge_index[0].astype(jnp.int32)
    col = edge_index[1].astype(jnp.int32)
    pad = jnp.full((epad - e,), n, jnp.int32)
    row_p = jnp.concatenate([row, pad])
    col_p = jnp.concatenate([col, pad])
    x_pad = jnp.pad(features, ((0, npn - n), (0, 0)))

    rowmap, deg_parts = _sc_degree_rowmap(row_p, col_p, n, npn, kc)
    deg4 = deg_parts.reshape(_NCORE, npn // 128, 1, 128)
    dinvb, z1 = _tc_prescale(deg4, x_pad)
    a1 = _sc_propagate(z1, rowmap, col_p, npn, kc)
    u1, z2 = _tc_mid(a1, dinvb)
    a2 = _sc_propagate(z2, rowmap, col_p, npn, kc)

    wa = W[0] - W[2]
    wb = W[1]
    wc = -2.0 * W[2]
    params = jnp.zeros((8, d), jnp.float32).at[0].set(b).at[1].set(gamma).at[2].set(beta)
    return _tc_final(features, u1, a2, dinvb, wa, wb, wc, params)
